# Initial kernel scaffold; baseline (speedup 1.0000x reference)
import jax
import jax.numpy as jnp
from jax.experimental import pallas as pl


def kernel(node_feat, edge, edge_feat, W1, b1, W2, b2, A1, a1, A2, a2, W_ih, b_ih, W_hh, b_hh):
    raise NotImplementedError("write your pallas kernel here")


# speedup vs baseline: 3.3242x; 3.3242x over previous
import functools

import jax
import jax.numpy as jnp
from jax import lax
from jax.experimental import pallas as pl
from jax.experimental.pallas import tpu as pltpu
from jax.experimental.pallas import tpu_sc as plsc

N = 10000
E = 320000
D = 128
DE = 16

NC = 2
NS = 16
NW = NC * NS
CH = 128
NCHUNK = E // CH
NPAD = 10112
ROWS_PER_SUB = NPAD // NS

@functools.cache
def _mesh():
    return plsc.VectorSubcoreMesh(
        core_axis_name="c", subcore_axis_name="s", num_cores=NC, num_subcores=NS
    )


def _gather_diff_body(src_hbm, dst_hbm, table_hbm, diff_hbm,
                      idx_s, idx_d, buf_a, buf_b, sem_a, sem_b):
    cid = lax.axis_index("c")
    sid = lax.axis_index("s")
    wid = sid * NC + cid
    lo = wid * NCHUNK // NW
    hi = (wid + 1) * NCHUNK // NW

    def step(k, carry):
        base = k * CH
        pltpu.sync_copy(src_hbm.at[pl.ds(base, CH)], idx_s.at[0])
        pltpu.sync_copy(dst_hbm.at[pl.ds(base, CH)], idx_d.at[0])
        cp_a = pltpu.async_copy(table_hbm.at[idx_s.at[0]], buf_a, sem_a)
        cp_b = pltpu.async_copy(table_hbm.at[idx_d.at[0]], buf_b, sem_b)
        cp_a.wait()
        cp_b.wait()

        def sub_row(r, c2):
            for j in range(D // 16):
                sl = pl.ds(j * 16, 16)
                buf_a[r, sl] = buf_a[r, sl] - buf_b[r, sl]
            return c2

        lax.fori_loop(0, CH, sub_row, 0)
        pltpu.sync_copy(buf_a, diff_hbm.at[pl.ds(base, CH), :])
        return carry

    lax.fori_loop(lo, hi, step, 0)


def _gather_diff(src, dst, node_feat):
    return pl.kernel(
        _gather_diff_body,
        out_type=jax.ShapeDtypeStruct((E, D), jnp.float32),
        mesh=_mesh(),
        scratch_types=[
            pltpu.VMEM((1, CH), jnp.int32),
            pltpu.VMEM((1, CH), jnp.int32),
            pltpu.VMEM((CH, D), jnp.float32),
            pltpu.VMEM((CH, D), jnp.float32),
            pltpu.SemaphoreType.DMA,
            pltpu.SemaphoreType.DMA,
        ],
    )(src, dst, node_feat)


BLK_E = 2560


def _edge_mlp_body(x_ref, f_ref, w1d, w1e, b1r, w2, b2r, a1d, a1e, a1r,
                   a2m, a2r, out_ref):
    x = x_ref[...]
    f = f_ref[...]
    dot = functools.partial(jnp.dot, preferred_element_type=jnp.float32)
    h1 = jnp.maximum(dot(x, w1d[...]) + dot(f, w1e[...]) + b1r[...], 0.0)
    msg = dot(h1, w2[...]) + b2r[...]
    ah = jnp.maximum(dot(x, a1d[...]) + dot(f, a1e[...]) + a1r[...], 0.0)
    att = jax.nn.sigmoid(dot(ah, a2m[...]) + a2r[...])
    out_ref[...] = msg * att


def _edge_mlp(diff, edge_feat, w1d, w1e, b1r, w2, b2r, a1d, a1e, a1r, a2m, a2r):
    full = lambda shape: pl.BlockSpec(shape, lambda i: (0, 0))
    return pl.pallas_call(
        _edge_mlp_body,
        grid=(E // BLK_E,),
        in_specs=[
            pl.BlockSpec((BLK_E, D), lambda i: (i, 0)),
            pl.BlockSpec((BLK_E, DE), lambda i: (i, 0)),
            full((D, D)), full((DE, D)), full((1, D)),
            full((D, D)), full((1, D)),
            full((D, D)), full((DE, D)), full((1, D)),
            full((D, D)), full((1, D)),
        ],
        out_specs=pl.BlockSpec((BLK_E, D), lambda i: (i, 0)),
        out_shape=jax.ShapeDtypeStruct((E, D), jnp.float32),
    )(diff, edge_feat, w1d, w1e, b1r, w2, b2r, a1d, a1e, a1r, a2m, a2r)


def _scatter_body(msg_hbm, dsti_hbm, zeros_hbm, out_hbm, idx, mbuf, accum, sem):
    cid = lax.axis_index("c")
    sid = lax.axis_index("s")
    wid = sid * NC + cid
    rbase = sid * ROWS_PER_SUB
    rows = pl.ds(rbase, ROWS_PER_SUB)
    pltpu.sync_copy(zeros_hbm.at[rows, :], accum.at[rows, :])
    plsc.subcore_barrier()

    lo = wid * NCHUNK // NW
    hi = (wid + 1) * NCHUNK // NW

    def step(k, carry):
        base = k * CH
        cp = pltpu.async_copy(msg_hbm.at[pl.ds(base, CH), :], mbuf, sem)
        pltpu.sync_copy(dsti_hbm.at[pl.ds(base, CH)], idx.at[0])
        cp.wait()
        pltpu.sync_copy(mbuf, accum.at[idx.at[0]], add=True)
        return carry

    lax.fori_loop(lo, hi, step, 0)
    plsc.subcore_barrier()
    pltpu.sync_copy(accum.at[rows, :], out_hbm.at[cid, rows, :])


def _scatter_add(msg, dst, zeros):
    return pl.kernel(
        _scatter_body,
        out_type=jax.ShapeDtypeStruct((NC, NPAD, D), jnp.float32),
        mesh=_mesh(),
        scratch_types=[
            pltpu.VMEM((1, CH), jnp.int32),
            pltpu.VMEM((CH, D), jnp.float32),
            pltpu.VMEM_SHARED((NPAD, D), jnp.float32),
            pltpu.SemaphoreType.DMA,
        ],
    )(msg, dst, zeros)


BLK_N = 2000


def _gru_body(p_ref, h_ref, wih, bih, whh, bhh, out_ref):
    sm = p_ref[0] + p_ref[1]
    h = h_ref[...]
    dot = functools.partial(jnp.dot, preferred_element_type=jnp.float32)
    gi = dot(sm, wih[...]) + bih[...]
    gh = dot(h, whh[...]) + bhh[...]
    r = jax.nn.sigmoid(gi[:, :D] + gh[:, :D])
    z = jax.nn.sigmoid(gi[:, D:2 * D] + gh[:, D:2 * D])
    n = jnp.tanh(gi[:, 2 * D:] + r * gh[:, 2 * D:])
    out_ref[...] = (1.0 - z) * n + z * h


def _gru(partials, h, wih, bih, whh, bhh):
    full = lambda shape: pl.BlockSpec(shape, lambda i: (0, 0))
    return pl.pallas_call(
        _gru_body,
        grid=(N // BLK_N,),
        in_specs=[
            pl.BlockSpec((NC, BLK_N, D), lambda i: (0, i, 0)),
            pl.BlockSpec((BLK_N, D), lambda i: (i, 0)),
            full((D, 3 * D)), full((1, 3 * D)),
            full((D, 3 * D)), full((1, 3 * D)),
        ],
        out_specs=pl.BlockSpec((BLK_N, D), lambda i: (i, 0)),
        out_shape=jax.ShapeDtypeStruct((N, D), jnp.float32),
    )(partials, h, wih, bih, whh, bhh)


def kernel(node_feat, edge, edge_feat, W1, b1, W2, b2, A1, a1, A2, a2,
           W_ih, b_ih, W_hh, b_hh):
    src = edge[:, 0]
    dst = edge[:, 1]
    diff = _gather_diff(src, dst, node_feat)
    msg = _edge_mlp(
        diff, edge_feat,
        W1[:, :D].T, W1[:, D:].T, b1[None],
        W2.T, b2[None],
        A1[:, :D].T, A1[:, D:].T, a1[None],
        A2.T, a2[None],
    )
    partials = _scatter_add(msg, dst, jnp.zeros((NPAD, D), jnp.float32))[:, :N]
    return _gru(partials, node_feat, W_ih.T, b_ih[None], W_hh.T, b_hh[None])


# speedup vs baseline: 3.5888x; 1.0796x over previous
import functools

import jax
import jax.numpy as jnp
from jax import lax
from jax.experimental import pallas as pl
from jax.experimental.pallas import tpu as pltpu
from jax.experimental.pallas import tpu_sc as plsc

N = 10000
E = 320000
D = 128
DE = 16

NC = 2
NS = 16
NW = NC * NS
EPW = E // NW
CHW = 80
CPW = EPW // CHW
NSLOT = 4
NSLOT_S = 2
NPAD = 10112
ROWS_PER_SUB = NPAD // NS

@functools.cache
def _mesh():
    return plsc.VectorSubcoreMesh(
        core_axis_name="c", subcore_axis_name="s", num_cores=NC, num_subcores=NS
    )


def _gather_diff_body(src_hbm, dst_hbm, table_hbm, diff_hbm,
                      idx_s, idx_d, buf_a, buf_b, sga, sgb, sout):
    cid = lax.axis_index("c")
    sid = lax.axis_index("s")
    wid = sid * NC + cid
    ebase = wid * EPW

    pltpu.sync_copy(src_hbm.at[wid], idx_s)
    pltpu.sync_copy(dst_hbm.at[wid], idx_d)

    def fire(j, b):
        pltpu.async_copy(table_hbm.at[idx_s.at[j]], buf_a.at[b], sga.at[b])
        pltpu.async_copy(table_hbm.at[idx_d.at[j]], buf_b.at[b], sgb.at[b])

    for j in range(NSLOT // 2):
        fire(j, j)

    def visit(k, carry):
        b = lax.rem(k, NSLOT)
        b2 = lax.rem(k + NSLOT // 2, NSLOT)

        @pl.when(k >= NSLOT // 2)
        def _drain_out():
            pltpu.make_async_copy(
                buf_a.at[b2], diff_hbm.at[pl.ds(0, CHW), :], sout.at[b2]
            ).wait()

        @pl.when(k + NSLOT // 2 < CPW)
        def _refill():
            fire(k + NSLOT // 2, b2)

        pltpu.make_async_copy(
            table_hbm.at[idx_s.at[k]], buf_a.at[b], sga.at[b]).wait()
        pltpu.make_async_copy(
            table_hbm.at[idx_d.at[k]], buf_b.at[b], sgb.at[b]).wait()

        def sub_row(r, c2):
            for j in range(D // 16):
                sl = pl.ds(j * 16, 16)
                buf_a[b, r, sl] = buf_a[b, r, sl] - buf_b[b, r, sl]
            return c2

        lax.fori_loop(0, CHW, sub_row, 0, unroll=2)
        pltpu.async_copy(
            buf_a.at[b], diff_hbm.at[pl.ds(ebase + k * CHW, CHW), :], sout.at[b])
        return carry

    lax.fori_loop(0, CPW, visit, 0)
    for kk in range(CPW - NSLOT // 2, CPW):
        pltpu.make_async_copy(
            buf_a.at[kk % NSLOT], diff_hbm.at[pl.ds(0, CHW), :],
            sout.at[kk % NSLOT]).wait()


def _gather_diff(src3, dst3, node_feat):
    return pl.kernel(
        _gather_diff_body,
        out_type=jax.ShapeDtypeStruct((E, D), jnp.float32),
        mesh=_mesh(),
        scratch_types=[
            pltpu.VMEM((CPW, CHW), jnp.int32),
            pltpu.VMEM((CPW, CHW), jnp.int32),
            pltpu.VMEM((NSLOT, CHW, D), jnp.float32),
            pltpu.VMEM((NSLOT, CHW, D), jnp.float32),
            pltpu.SemaphoreType.DMA((NSLOT,)),
            pltpu.SemaphoreType.DMA((NSLOT,)),
            pltpu.SemaphoreType.DMA((NSLOT,)),
        ],
    )(src3, dst3, node_feat)


BLK_E = 2560


def _edge_mlp_body(x_ref, f_ref, w1d, w1e, b1r, w2, b2r, a1d, a1e, a1r,
                   a2m, a2r, out_ref):
    x = x_ref[...]
    f = f_ref[...]
    dot = functools.partial(jnp.dot, preferred_element_type=jnp.float32)
    h1 = jnp.maximum(dot(x, w1d[...]) + dot(f, w1e[...]) + b1r[...], 0.0)
    msg = dot(h1, w2[...]) + b2r[...]
    ah = jnp.maximum(dot(x, a1d[...]) + dot(f, a1e[...]) + a1r[...], 0.0)
    att = jax.nn.sigmoid(dot(ah, a2m[...]) + a2r[...])
    out_ref[...] = msg * att


def _edge_mlp(diff, edge_feat, w1d, w1e, b1r, w2, b2r, a1d, a1e, a1r, a2m, a2r):
    full = lambda shape: pl.BlockSpec(shape, lambda i: (0, 0))
    return pl.pallas_call(
        _edge_mlp_body,
        grid=(E // BLK_E,),
        in_specs=[
            pl.BlockSpec((BLK_E, D), lambda i: (i, 0)),
            pl.BlockSpec((BLK_E, DE), lambda i: (i, 0)),
            full((D, D)), full((DE, D)), full((1, D)),
            full((D, D)), full((1, D)),
            full((D, D)), full((DE, D)), full((1, D)),
            full((D, D)), full((1, D)),
        ],
        out_specs=pl.BlockSpec((BLK_E, D), lambda i: (i, 0)),
        out_shape=jax.ShapeDtypeStruct((E, D), jnp.float32),
    )(diff, edge_feat, w1d, w1e, b1r, w2, b2r, a1d, a1e, a1r, a2m, a2r)


def _scatter_body(msg_hbm, dsti_hbm, zeros_hbm, out_hbm,
                  idx, mbuf, accum, srd, ssc):
    cid = lax.axis_index("c")
    sid = lax.axis_index("s")
    wid = sid * NC + cid
    ebase = wid * EPW
    rbase = sid * ROWS_PER_SUB
    rows = pl.ds(rbase, ROWS_PER_SUB)
    pltpu.sync_copy(zeros_hbm.at[rows, :], accum.at[rows, :])
    pltpu.sync_copy(dsti_hbm.at[wid], idx)
    plsc.subcore_barrier()

    def fire(j, b):
        pltpu.async_copy(
            msg_hbm.at[pl.ds(ebase + j * CHW, CHW), :], mbuf.at[b], srd.at[b])

    for j in range(NSLOT_S // 2):
        fire(j, j)

    def visit(k, carry):
        b = lax.rem(k, NSLOT_S)
        b2 = lax.rem(k + NSLOT_S // 2, NSLOT_S)

        @pl.when(k >= NSLOT_S // 2)
        def _drain_sc():
            pltpu.make_async_copy(
                mbuf.at[b2], accum.at[idx.at[k]], ssc.at[b2]).wait()

        @pl.when(k + NSLOT_S // 2 < CPW)
        def _refill():
            fire(k + NSLOT_S // 2, b2)

        pltpu.make_async_copy(
            msg_hbm.at[pl.ds(0, CHW), :], mbuf.at[b], srd.at[b]).wait()
        pltpu.async_copy(mbuf.at[b], accum.at[idx.at[k]], ssc.at[b], add=True)
        return carry

    lax.fori_loop(0, CPW, visit, 0)
    for kk in range(CPW - NSLOT_S // 2, CPW):
        pltpu.make_async_copy(
            mbuf.at[kk % NSLOT_S], accum.at[idx.at[kk]],
            ssc.at[kk % NSLOT_S]).wait()
    plsc.subcore_barrier()
    pltpu.sync_copy(accum.at[rows, :], out_hbm.at[cid, rows, :])


def _scatter_add(msg, dst3, zeros):
    return pl.kernel(
        _scatter_body,
        out_type=jax.ShapeDtypeStruct((NC, NPAD, D), jnp.float32),
        mesh=_mesh(),
        scratch_types=[
            pltpu.VMEM((CPW, CHW), jnp.int32),
            pltpu.VMEM((NSLOT_S, CHW, D), jnp.float32),
            pltpu.VMEM_SHARED((NPAD, D), jnp.float32),
            pltpu.SemaphoreType.DMA((NSLOT_S,)),
            pltpu.SemaphoreType.DMA((NSLOT_S,)),
        ],
    )(msg, dst3, zeros)


BLK_N = 2000


def _gru_body(p_ref, h_ref, wih, bih, whh, bhh, out_ref):
    sm = p_ref[0] + p_ref[1]
    h = h_ref[...]
    dot = functools.partial(jnp.dot, preferred_element_type=jnp.float32)
    gi = dot(sm, wih[...]) + bih[...]
    gh = dot(h, whh[...]) + bhh[...]
    r = jax.nn.sigmoid(gi[:, :D] + gh[:, :D])
    z = jax.nn.sigmoid(gi[:, D:2 * D] + gh[:, D:2 * D])
    n = jnp.tanh(gi[:, 2 * D:] + r * gh[:, 2 * D:])
    out_ref[...] = (1.0 - z) * n + z * h


def _gru(partials, h, wih, bih, whh, bhh):
    full = lambda shape: pl.BlockSpec(shape, lambda i: (0, 0))
    return pl.pallas_call(
        _gru_body,
        grid=(N // BLK_N,),
        in_specs=[
            pl.BlockSpec((NC, BLK_N, D), lambda i: (0, i, 0)),
            pl.BlockSpec((BLK_N, D), lambda i: (i, 0)),
            full((D, 3 * D)), full((1, 3 * D)),
            full((D, 3 * D)), full((1, 3 * D)),
        ],
        out_specs=pl.BlockSpec((BLK_N, D), lambda i: (i, 0)),
        out_shape=jax.ShapeDtypeStruct((N, D), jnp.float32),
    )(partials, h, wih, bih, whh, bhh)


def kernel(node_feat, edge, edge_feat, W1, b1, W2, b2, A1, a1, A2, a2,
           W_ih, b_ih, W_hh, b_hh):
    src3 = edge[:, 0].reshape(NW, CPW, CHW)
    dst3 = edge[:, 1].reshape(NW, CPW, CHW)
    diff = _gather_diff(src3, dst3, node_feat)
    msg = _edge_mlp(
        diff, edge_feat,
        W1[:, :D].T, W1[:, D:].T, b1[None],
        W2.T, b2[None],
        A1[:, :D].T, A1[:, D:].T, a1[None],
        A2.T, a2[None],
    )
    partials = _scatter_add(msg, dst3, jnp.zeros((NPAD, D), jnp.float32))[:, :N]
    return _gru(partials, node_feat, W_ih.T, b_ih[None], W_hh.T, b_hh[None])


# speedup vs baseline: 4.3233x; 1.2047x over previous
import functools

import jax
import jax.numpy as jnp
from jax import lax
from jax.experimental import pallas as pl
from jax.experimental.pallas import tpu as pltpu
from jax.experimental.pallas import tpu_sc as plsc

N = 10000
E = 320000
D = 128
DE = 16

NC = 2
NS = 16
NW = NC * NS
EPW = E // NW
CHW = 80
CPW = EPW // CHW
NSLOT = 4
NSLOT_S = 2
NPAD = 10112
ROWS_PER_SUB = NPAD // NS

@functools.cache
def _mesh():
    return plsc.VectorSubcoreMesh(
        core_axis_name="c", subcore_axis_name="s", num_cores=NC, num_subcores=NS
    )


def _gather_diff_body(src_hbm, dst_hbm, table_hbm, ntable_hbm, diff_hbm,
                      idx_s, idx_d, buf, sga, sgb, sout):
    cid = lax.axis_index("c")
    sid = lax.axis_index("s")
    wid = sid * NC + cid
    ebase = wid * EPW

    pltpu.sync_copy(src_hbm.at[wid], idx_s)
    pltpu.sync_copy(dst_hbm.at[wid], idx_d)

    def fire1(j, b):
        pltpu.async_copy(table_hbm.at[idx_s.at[j]], buf.at[b], sga.at[b])

    def fire2(j, b):
        pltpu.async_copy(ntable_hbm.at[idx_d.at[j]], buf.at[b], sgb.at[b],
                         add=True)

    fire1(0, 0)
    fire1(1, 1)
    pltpu.make_async_copy(table_hbm.at[idx_s.at[0]], buf.at[0], sga.at[0]).wait()
    fire2(0, 0)

    def visit(k, carry):
        b = lax.rem(k, NSLOT)
        b1 = lax.rem(k + 1, NSLOT)
        b2 = lax.rem(k + NSLOT // 2, NSLOT)

        @pl.when(k >= NSLOT // 2)
        def _drain_out():
            pltpu.make_async_copy(
                buf.at[b2], diff_hbm.at[pl.ds(0, CHW), :], sout.at[b2]).wait()

        @pl.when(k + NSLOT // 2 < CPW)
        def _fire_g1():
            fire1(k + NSLOT // 2, b2)

        @pl.when(k + 1 < CPW)
        def _fire_g2():
            pltpu.make_async_copy(
                table_hbm.at[idx_s.at[k]], buf.at[b1], sga.at[b1]).wait()
            fire2(k + 1, b1)

        pltpu.make_async_copy(
            ntable_hbm.at[idx_d.at[k]], buf.at[b], sgb.at[b]).wait()
        pltpu.async_copy(
            buf.at[b], diff_hbm.at[pl.ds(ebase + k * CHW, CHW), :], sout.at[b])
        return carry

    lax.fori_loop(0, CPW, visit, 0)
    for kk in range(CPW - NSLOT // 2, CPW):
        pltpu.make_async_copy(
            buf.at[kk % NSLOT], diff_hbm.at[pl.ds(0, CHW), :],
            sout.at[kk % NSLOT]).wait()


def _gather_diff(src3, dst3, node_feat, neg_node_feat):
    return pl.kernel(
        _gather_diff_body,
        out_type=jax.ShapeDtypeStruct((E, D), jnp.float32),
        mesh=_mesh(),
        scratch_types=[
            pltpu.VMEM((CPW, CHW), jnp.int32),
            pltpu.VMEM((CPW, CHW), jnp.int32),
            pltpu.VMEM((NSLOT, CHW, D), jnp.float32),
            pltpu.SemaphoreType.DMA((NSLOT,)),
            pltpu.SemaphoreType.DMA((NSLOT,)),
            pltpu.SemaphoreType.DMA((NSLOT,)),
        ],
    )(src3, dst3, node_feat, neg_node_feat)


BLK_E = 2560


def _edge_mlp_body(x_ref, f_ref, w1d, w1e, b1r, w2, b2r, a1d, a1e, a1r,
                   a2m, a2r, out_ref):
    x = x_ref[...]
    f = f_ref[...]
    dot = functools.partial(jnp.dot, preferred_element_type=jnp.float32)
    h1 = jnp.maximum(dot(x, w1d[...]) + dot(f, w1e[...]) + b1r[...], 0.0)
    msg = dot(h1, w2[...]) + b2r[...]
    ah = jnp.maximum(dot(x, a1d[...]) + dot(f, a1e[...]) + a1r[...], 0.0)
    att = jax.nn.sigmoid(dot(ah, a2m[...]) + a2r[...])
    out_ref[...] = msg * att


def _edge_mlp(diff, edge_feat, w1d, w1e, b1r, w2, b2r, a1d, a1e, a1r, a2m, a2r):
    full = lambda shape: pl.BlockSpec(shape, lambda i: (0, 0))
    return pl.pallas_call(
        _edge_mlp_body,
        grid=(E // BLK_E,),
        in_specs=[
            pl.BlockSpec((BLK_E, D), lambda i: (i, 0)),
            pl.BlockSpec((BLK_E, DE), lambda i: (i, 0)),
            full((D, D)), full((DE, D)), full((1, D)),
            full((D, D)), full((1, D)),
            full((D, D)), full((DE, D)), full((1, D)),
            full((D, D)), full((1, D)),
        ],
        out_specs=pl.BlockSpec((BLK_E, D), lambda i: (i, 0)),
        out_shape=jax.ShapeDtypeStruct((E, D), jnp.float32),
    )(diff, edge_feat, w1d, w1e, b1r, w2, b2r, a1d, a1e, a1r, a2m, a2r)


def _scatter_body(msg_hbm, dsti_hbm, zeros_hbm, out_hbm,
                  idx, mbuf, accum, srd, ssc):
    cid = lax.axis_index("c")
    sid = lax.axis_index("s")
    wid = sid * NC + cid
    ebase = wid * EPW
    rbase = sid * ROWS_PER_SUB
    rows = pl.ds(rbase, ROWS_PER_SUB)
    pltpu.sync_copy(zeros_hbm.at[rows, :], accum.at[rows, :])
    pltpu.sync_copy(dsti_hbm.at[wid], idx)
    plsc.subcore_barrier()

    def fire(j, b):
        pltpu.async_copy(
            msg_hbm.at[pl.ds(ebase + j * CHW, CHW), :], mbuf.at[b], srd.at[b])

    for j in range(NSLOT_S // 2):
        fire(j, j)

    def visit(k, carry):
        b = lax.rem(k, NSLOT_S)
        b2 = lax.rem(k + NSLOT_S // 2, NSLOT_S)

        @pl.when(k >= NSLOT_S // 2)
        def _drain_sc():
            pltpu.make_async_copy(
                mbuf.at[b2], accum.at[idx.at[k]], ssc.at[b2]).wait()

        @pl.when(k + NSLOT_S // 2 < CPW)
        def _refill():
            fire(k + NSLOT_S // 2, b2)

        pltpu.make_async_copy(
            msg_hbm.at[pl.ds(0, CHW), :], mbuf.at[b], srd.at[b]).wait()
        pltpu.async_copy(mbuf.at[b], accum.at[idx.at[k]], ssc.at[b], add=True)
        return carry

    lax.fori_loop(0, CPW, visit, 0)
    for kk in range(CPW - NSLOT_S // 2, CPW):
        pltpu.make_async_copy(
            mbuf.at[kk % NSLOT_S], accum.at[idx.at[kk]],
            ssc.at[kk % NSLOT_S]).wait()
    plsc.subcore_barrier()
    pltpu.sync_copy(accum.at[rows, :], out_hbm.at[cid, rows, :])


def _scatter_add(msg, dst3, zeros):
    return pl.kernel(
        _scatter_body,
        out_type=jax.ShapeDtypeStruct((NC, NPAD, D), jnp.float32),
        mesh=_mesh(),
        scratch_types=[
            pltpu.VMEM((CPW, CHW), jnp.int32),
            pltpu.VMEM((NSLOT_S, CHW, D), jnp.float32),
            pltpu.VMEM_SHARED((NPAD, D), jnp.float32),
            pltpu.SemaphoreType.DMA((NSLOT_S,)),
            pltpu.SemaphoreType.DMA((NSLOT_S,)),
        ],
    )(msg, dst3, zeros)


BLK_N = 2000


def _gru_body(p_ref, h_ref, wih, bih, whh, bhh, out_ref):
    sm = p_ref[0] + p_ref[1]
    h = h_ref[...]
    dot = functools.partial(jnp.dot, preferred_element_type=jnp.float32)
    gi = dot(sm, wih[...]) + bih[...]
    gh = dot(h, whh[...]) + bhh[...]
    r = jax.nn.sigmoid(gi[:, :D] + gh[:, :D])
    z = jax.nn.sigmoid(gi[:, D:2 * D] + gh[:, D:2 * D])
    n = jnp.tanh(gi[:, 2 * D:] + r * gh[:, 2 * D:])
    out_ref[...] = (1.0 - z) * n + z * h


def _gru(partials, h, wih, bih, whh, bhh):
    full = lambda shape: pl.BlockSpec(shape, lambda i: (0, 0))
    return pl.pallas_call(
        _gru_body,
        grid=(N // BLK_N,),
        in_specs=[
            pl.BlockSpec((NC, BLK_N, D), lambda i: (0, i, 0)),
            pl.BlockSpec((BLK_N, D), lambda i: (i, 0)),
            full((D, 3 * D)), full((1, 3 * D)),
            full((D, 3 * D)), full((1, 3 * D)),
        ],
        out_specs=pl.BlockSpec((BLK_N, D), lambda i: (i, 0)),
        out_shape=jax.ShapeDtypeStruct((N, D), jnp.float32),
    )(partials, h, wih, bih, whh, bhh)


def kernel(node_feat, edge, edge_feat, W1, b1, W2, b2, A1, a1, A2, a2,
           W_ih, b_ih, W_hh, b_hh):
    src3 = edge[:, 0].reshape(NW, CPW, CHW)
    dst3 = edge[:, 1].reshape(NW, CPW, CHW)
    diff = _gather_diff(src3, dst3, node_feat, jnp.negative(node_feat))
    msg = _edge_mlp(
        diff, edge_feat,
        W1[:, :D].T, W1[:, D:].T, b1[None],
        W2.T, b2[None],
        A1[:, :D].T, A1[:, D:].T, a1[None],
        A2.T, a2[None],
    )
    partials = _scatter_add(msg, dst3, jnp.zeros((NPAD, D), jnp.float32))[:, :N]
    return _gru(partials, node_feat, W_ih.T, b_ih[None], W_hh.T, b_hh[None])
